# baseline (speedup 1.0000x reference)
import jax
import jax.numpy as jnp
from jax.experimental import pallas as pl
from jax.experimental.pallas import tpu as pltpu

_MAXLEN = 8192
_EMBED = 128
_BATCH = 4
_ROWS = _BATCH * _MAXLEN
_CHUNK = 2048
_N = _ROWS // _CHUNK
_PPARTS = _MAXLEN // _CHUNK
_RING = 4


def _body(x_hbm, pos_hbm, out_hbm, pos_v, xbufs, obufs, xsems, osems, psems):
    ploads = [
        pltpu.async_copy(
            pos_hbm.at[pl.ds(p * _CHUNK, _CHUNK), :],
            pos_v.at[pl.ds(p * _CHUNK, _CHUNK), :], psems.at[p])
        for p in range(_PPARTS)
    ]
    xloads = {}
    for c in range(_RING):
        xloads[c] = pltpu.async_copy(
            x_hbm.at[pl.ds(c * _CHUNK, _CHUNK), :], xbufs.at[c],
            xsems.at[c])

    stores = {}
    for c in range(_N):
        slot = c % _RING
        xloads[c].wait()
        if c < _PPARTS:
            ploads[c].wait()
        if c >= _RING:
            stores[c - _RING].wait()
        p = c % _PPARTS
        obufs[slot] = xbufs[slot] + pos_v[pl.ds(p * _CHUNK, _CHUNK), :]
        stores[c] = pltpu.async_copy(
            obufs.at[slot], out_hbm.at[pl.ds(c * _CHUNK, _CHUNK), :],
            osems.at[slot])
        if c + _RING < _N:
            xloads[c + _RING] = pltpu.async_copy(
                x_hbm.at[pl.ds((c + _RING) * _CHUNK, _CHUNK), :],
                xbufs.at[slot], xsems.at[slot])
    for c in range(_N - _RING, _N):
        stores[c].wait()


def kernel(x, pos_table):
    out = pl.pallas_call(
        _body,
        in_specs=[
            pl.BlockSpec(memory_space=pltpu.HBM),
            pl.BlockSpec(memory_space=pltpu.HBM),
        ],
        out_specs=pl.BlockSpec(memory_space=pltpu.HBM),
        out_shape=jax.ShapeDtypeStruct((_ROWS, _EMBED), jnp.float32),
        scratch_shapes=[
            pltpu.VMEM((_MAXLEN, _EMBED), jnp.float32),
            pltpu.VMEM((_RING, _CHUNK, _EMBED), jnp.float32),
            pltpu.VMEM((_RING, _CHUNK, _EMBED), jnp.float32),
            pltpu.SemaphoreType.DMA((_RING,)),
            pltpu.SemaphoreType.DMA((_RING,)),
            pltpu.SemaphoreType.DMA((_PPARTS,)),
        ],
    )(x.reshape(_ROWS, _EMBED), pos_table)
    return out.reshape(_BATCH, _MAXLEN, _EMBED)


# speedup vs baseline: 1.0819x; 1.0819x over previous
import jax
import jax.numpy as jnp
from jax.experimental import pallas as pl

_MAXLEN = 8192
_EMBED = 128
_BATCH = 4
_BBLK = 2


def _add_body(x_ref, p_ref, o_ref):
    o_ref[...] = x_ref[...] + p_ref[...][None, :, :]


def kernel(x, pos_table):
    return pl.pallas_call(
        _add_body,
        grid=(_BATCH // _BBLK,),
        in_specs=[
            pl.BlockSpec((_BBLK, _MAXLEN, _EMBED), lambda b: (b, 0, 0)),
            pl.BlockSpec((_MAXLEN, _EMBED), lambda b: (0, 0)),
        ],
        out_specs=pl.BlockSpec((_BBLK, _MAXLEN, _EMBED), lambda b: (b, 0, 0)),
        out_shape=jax.ShapeDtypeStruct((_BATCH, _MAXLEN, _EMBED), jnp.float32),
    )(x, pos_table)
